# baseline (speedup 1.0000x reference)
import jax
import jax.numpy as jnp
from jax.experimental import pallas as pl

B = 4096
S = 8
D = 512
P = 256
EMB = 64
NE = 7

BB = 512


def _gru_block(states_ref, ids_ref, du_ref, emo_ref, emb_ref, w_ih_ref,
               w_hh_ref, b_ih_ref, b_hh_ref, out_ref):
    ids = ids_ref[...]
    emo = emo_ref[...]

    h_old = states_ref[:, 0:D]
    for s in range(1, S):
        h_old = jnp.where(ids == s, states_ref[:, s * D:(s + 1) * D], h_old)

    emask = (emo == jax.lax.broadcasted_iota(jnp.int32, (BB, NE + 1), 1))
    other_emb = jax.lax.dot_general(
        emask.astype(jnp.float32), emb_ref[...],
        (((1,), (0,)), ((), ())), preferred_element_type=jnp.float32)

    gi = jax.lax.dot_general(du_ref[...], w_ih_ref[:, :P],
                             (((1,), (1,)), ((), ())),
                             preferred_element_type=jnp.float32)
    gi += jax.lax.dot_general(other_emb, w_ih_ref[:, P:],
                              (((1,), (1,)), ((), ())),
                              preferred_element_type=jnp.float32)
    gi += b_ih_ref[...]
    gh = jax.lax.dot_general(h_old, w_hh_ref[...],
                             (((1,), (1,)), ((), ())),
                             preferred_element_type=jnp.float32)
    gh += b_hh_ref[...]

    r = jax.nn.sigmoid(gi[:, :D] + gh[:, :D])
    z = jax.nn.sigmoid(gi[:, D:2 * D] + gh[:, D:2 * D])
    n = jnp.tanh(gi[:, 2 * D:] + r * gh[:, 2 * D:])
    h_new = (1.0 - z) * n + z * h_old

    for s in range(S):
        out_ref[:, s * D:(s + 1) * D] = jnp.where(
            ids == s, h_new, states_ref[:, s * D:(s + 1) * D])


def kernel(states, speaker_ids, delta_u, other_emo_ids, emb_table, w_ih,
           w_hh, b_ih, b_hh):
    states2 = states.reshape(B, S * D)
    ids2 = jnp.clip(speaker_ids, 0, S - 1).astype(jnp.int32).reshape(B, 1)
    emo2 = other_emo_ids.astype(jnp.int32).reshape(B, 1)
    b_ih2 = b_ih.reshape(1, 3 * D)
    b_hh2 = b_hh.reshape(1, 3 * D)

    grid = (B // BB,)
    out = pl.pallas_call(
        _gru_block,
        grid=grid,
        in_specs=[
            pl.BlockSpec((BB, S * D), lambda i: (i, 0)),
            pl.BlockSpec((BB, 1), lambda i: (i, 0)),
            pl.BlockSpec((BB, P), lambda i: (i, 0)),
            pl.BlockSpec((BB, 1), lambda i: (i, 0)),
            pl.BlockSpec((NE + 1, EMB), lambda i: (0, 0)),
            pl.BlockSpec((3 * D, P + EMB), lambda i: (0, 0)),
            pl.BlockSpec((3 * D, D), lambda i: (0, 0)),
            pl.BlockSpec((1, 3 * D), lambda i: (0, 0)),
            pl.BlockSpec((1, 3 * D), lambda i: (0, 0)),
        ],
        out_specs=pl.BlockSpec((BB, S * D), lambda i: (i, 0)),
        out_shape=jax.ShapeDtypeStruct((B, S * D), states.dtype),
    )(states2, ids2, delta_u, emo2, emb_table, w_ih, w_hh, b_ih2, b_hh2)
    return out.reshape(B, S, D)


# speedup vs baseline: 2.3084x; 2.3084x over previous
import jax
import jax.numpy as jnp
from jax.experimental import pallas as pl

B = 4096
S = 8
D = 512
P = 256
EMB = 64
NE = 7

BB = 512


def _gru_block(states_ref, ids_ref, du_ref, emo_ref, emb_ref, w_ih_ref,
               w_hh_ref, b_ih_ref, b_hh_ref, out_ref):
    ids3 = ids_ref[...]
    emo = emo_ref[...]

    st = states_ref[...]
    iota_s = jax.lax.broadcasted_iota(jnp.int32, (BB, S, D), 1)
    mask3 = ids3 == iota_s
    h_old = jnp.sum(jnp.where(mask3, st, 0.0), axis=1)

    emask = (emo == jax.lax.broadcasted_iota(jnp.int32, (BB, NE + 1), 1))
    other_emb = jax.lax.dot_general(
        emask.astype(jnp.float32), emb_ref[...],
        (((1,), (0,)), ((), ())), preferred_element_type=jnp.float32)

    gi = jax.lax.dot_general(du_ref[...], w_ih_ref[:, :P],
                             (((1,), (1,)), ((), ())),
                             preferred_element_type=jnp.float32)
    gi += jax.lax.dot_general(other_emb, w_ih_ref[:, P:],
                              (((1,), (1,)), ((), ())),
                              preferred_element_type=jnp.float32)
    gi += b_ih_ref[...]
    gh = jax.lax.dot_general(h_old, w_hh_ref[...],
                             (((1,), (1,)), ((), ())),
                             preferred_element_type=jnp.float32)
    gh += b_hh_ref[...]

    r = jax.nn.sigmoid(gi[:, :D] + gh[:, :D])
    z = jax.nn.sigmoid(gi[:, D:2 * D] + gh[:, D:2 * D])
    n = jnp.tanh(gi[:, 2 * D:] + r * gh[:, 2 * D:])
    h_new = (1.0 - z) * n + z * h_old

    h_new3 = jax.lax.broadcast_in_dim(h_new, (BB, S, D), (0, 2))
    out_ref[...] = jnp.where(mask3, h_new3, st)


def kernel(states, speaker_ids, delta_u, other_emo_ids, emb_table, w_ih,
           w_hh, b_ih, b_hh):
    ids3 = jnp.clip(speaker_ids, 0, S - 1).astype(jnp.int32).reshape(B, 1, 1)
    emo2 = other_emo_ids.astype(jnp.int32).reshape(B, 1)
    b_ih2 = b_ih.reshape(1, 3 * D)
    b_hh2 = b_hh.reshape(1, 3 * D)

    grid = (B // BB,)
    out = pl.pallas_call(
        _gru_block,
        grid=grid,
        in_specs=[
            pl.BlockSpec((BB, S, D), lambda i: (i, 0, 0)),
            pl.BlockSpec((BB, 1, 1), lambda i: (i, 0, 0)),
            pl.BlockSpec((BB, P), lambda i: (i, 0)),
            pl.BlockSpec((BB, 1), lambda i: (i, 0)),
            pl.BlockSpec((NE + 1, EMB), lambda i: (0, 0)),
            pl.BlockSpec((3 * D, P + EMB), lambda i: (0, 0)),
            pl.BlockSpec((3 * D, D), lambda i: (0, 0)),
            pl.BlockSpec((1, 3 * D), lambda i: (0, 0)),
            pl.BlockSpec((1, 3 * D), lambda i: (0, 0)),
        ],
        out_specs=pl.BlockSpec((BB, S, D), lambda i: (i, 0, 0)),
        out_shape=jax.ShapeDtypeStruct((B, S, D), states.dtype),
    )(states, ids3, delta_u, emo2, emb_table, w_ih, w_hh, b_ih2, b_hh2)
    return out
